# baseline (speedup 1.0000x reference)
import functools

import jax
import jax.numpy as jnp
from jax import lax
from jax.experimental import pallas as pl
from jax.experimental.pallas import tpu as pltpu
from jax.experimental.pallas import tpu_sc as plsc

_NUM_CLASSES = 81
_NUM_KNOWN = 80
_ALPHA_PROTO = 0.9
_TAU_CEC = 0.1
_FOCAL_ALPHA = 0.25
_W_CE = 2.0
_W_CEC = 0.5

_B, _Q, _D, _T = 32, 900, 256, 20
_N = _B * _T
_TP = 24
_NP = _B * _TP
_TX = 32
_NX = _B * _TX
_CCH = 9


def _make_sc_gather():
  mesh = plsc.VectorSubcoreMesh(core_axis_name="c", subcore_axis_name="s")
  info = plsc.get_sparse_core_info()
  num_cores = info.num_cores

  @functools.partial(
      pl.kernel,
      mesh=mesh,
      out_type=jax.ShapeDtypeStruct((_NP, _D), jnp.float32),
      scratch_types=[
          pltpu.VMEM((_TP,), jnp.int32),
          pltpu.VMEM((_TP, _D), jnp.float32),
          pltpu.SemaphoreType.DMA,
      ],
  )
  def sc_gather(eidx_hbm, table_hbm, emb_out, eidx_v, erows_v, sem_e):
    wid = lax.axis_index("s") * num_cores + lax.axis_index("c")
    ebase = pl.multiple_of(wid * _TP, 8)
    pltpu.sync_copy(eidx_hbm.at[pl.ds(ebase, _TP)], eidx_v)
    pltpu.async_copy(table_hbm.at[eidx_v], erows_v, sem_e).wait()
    pltpu.sync_copy(erows_v, emb_out.at[pl.ds(ebase, _TP)])

  return sc_gather


_sc_gather_cache = []


def _get_sc_gather():
  if not _sc_gather_cache:
    _sc_gather_cache.append(_make_sc_gather())
  return _sc_gather_cache[0]


_BCH = 16


def _dense_body(x_ref, si_ref, lb_ref, sv_ref, out_ref, acc_ref):
  i = pl.program_id(0)
  x = x_ref[...]

  a = jnp.exp(-jnp.abs(x))
  lg = jnp.log(1.0 + a)
  r = 1.0 / (1.0 + a)
  p = jnp.where(x >= 0.0, r, 1.0 - r)
  sp = jnp.maximum(x, 0.0) + lg
  s0 = jnp.sum(p * p * sp)

  q_iota = lax.broadcasted_iota(jnp.int32, (1, _Q), 1)
  t_iota = lax.broadcasted_iota(jnp.int32, (_BCH, 128), 1)
  b_iota = lax.broadcasted_iota(jnp.int32, (_BCH, 128), 0)
  xs = jnp.zeros((_BCH, 128), jnp.float32)
  for bb in range(_BCH):
    b = i * _BCH + bb
    sts = [si_ref[b, t] for t in range(_T)]
    lts = [lb_ref[b, t] for t in range(_T)]
    for t in range(_T):
      row = x_ref[pl.ds(lts[t], 1), pl.ds(bb, 1), :].reshape(1, _Q)
      xt = jnp.sum(jnp.where(q_iota == sts[t], row, 0.0))
      m2 = jnp.logical_and(t_iota == t, b_iota == bb)
      xs = jnp.where(m2, xt, xs)

  src = sv_ref[...]
  dup = jnp.zeros((_BCH, _T), jnp.float32)
  for d in range(1, _T):
    eq = (src[:, : _T - d] == src[:, d:]).astype(jnp.float32)
    eq = jnp.concatenate([eq, jnp.zeros((_BCH, d), jnp.float32)], axis=1)
    dup = jnp.maximum(dup, eq)
  fin = 1.0 - dup

  a2 = jnp.exp(-jnp.abs(xs))
  lg2 = jnp.log(1.0 + a2)
  r2 = 1.0 / (1.0 + a2)
  p2 = jnp.where(xs >= 0.0, r2, 1.0 - r2)
  sp2 = jnp.maximum(xs, 0.0) + lg2
  sn2 = sp2 - xs
  om = 1.0 - p2
  g = _FOCAL_ALPHA * om * om * sn2 - (1.0 - _FOCAL_ALPHA) * p2 * p2 * sp2
  part = (1.0 - _FOCAL_ALPHA) * s0 + jnp.sum(fin * g[:, :_T])

  @pl.when(i == 0)
  def _():
    acc_ref[0, 0] = 0.0

  acc = acc_ref[0, 0] + part
  acc_ref[0, 0] = acc

  @pl.when(i == _B // _BCH - 1)
  def _():
    out_ref[0, 0] = acc


def _dense_focal(pred_cm, src_idx, labels):
  return pl.pallas_call(
      _dense_body,
      grid=(_B // _BCH,),
      in_specs=[
          pl.BlockSpec((_NUM_CLASSES, _BCH, _Q), lambda i: (0, i, 0)),
          pl.BlockSpec(memory_space=pltpu.SMEM),
          pl.BlockSpec(memory_space=pltpu.SMEM),
          pl.BlockSpec((_BCH, _T), lambda i: (i, 0)),
      ],
      out_specs=pl.BlockSpec(memory_space=pltpu.SMEM),
      out_shape=jax.ShapeDtypeStruct((1, 1), jnp.float32),
      scratch_shapes=[pltpu.SMEM((1, 1), jnp.float32)],
      compiler_params=pltpu.CompilerParams(
          dimension_semantics=("arbitrary",)),
  )(pred_cm, src_idx, labels, src_idx)


def _combine_body(s0_ref, lf_ref, g_ref, cm_ref, out_ref):
  num_boxes = jnp.maximum(jnp.float32(_N), 1.0)
  loss_ce = s0_ref[0, 0] / num_boxes

  embs = g_ref[...]
  nrm = jnp.sqrt(jnp.sum(embs * embs, axis=1, keepdims=True))
  embs_n = embs / jnp.maximum(nrm, 1e-6)

  lf = lf_ref[...]
  valid_col = lf >= 0
  k_iota = lax.broadcasted_iota(jnp.int32, (_NUM_KNOWN, _NP), 0)
  mask = (lf == k_iota).astype(jnp.float32)
  counts = jnp.sum(mask, axis=1, keepdims=True)

  sums = lax.dot_general(mask, embs, (((1,), (0,)), ((), ())),
                         preferred_element_type=jnp.float32)
  mean = sums / jnp.maximum(counts, 1.0)
  mnrm = jnp.sqrt(jnp.sum(mean * mean, axis=1, keepdims=True))
  mean_n = mean / jnp.maximum(mnrm, 1e-6)

  cm = cm_ref[...]
  upd = _ALPHA_PROTO * cm + (1.0 - _ALPHA_PROTO) * mean_n
  unrm = jnp.sqrt(jnp.sum(upd * upd, axis=1, keepdims=True))
  upd_n = upd / jnp.maximum(unrm, 1e-6)
  protos = jnp.where(counts > 0, upd_n, cm)
  pnrm = jnp.sqrt(jnp.sum(protos * protos, axis=1, keepdims=True))
  protos_n = protos / jnp.maximum(pnrm, 1e-6)

  sim = lax.dot_general(protos_n, embs_n, (((1,), (1,)), ((), ())),
                        preferred_element_type=jnp.float32)
  logits = jnp.where(valid_col, sim / _TAU_CEC, -1e30)
  m = jnp.max(logits, axis=1, keepdims=True)
  lse = jnp.log(jnp.sum(jnp.exp(logits - m), axis=1, keepdims=True)) + m
  logp = logits - lse
  per_class = -jnp.sum(logp * mask, axis=1, keepdims=True)
  per_class = per_class / jnp.maximum(counts, 1.0)
  validm = (counts > 0).astype(jnp.float32)
  cec = jnp.sum(per_class * validm) / jnp.maximum(jnp.sum(validm), 1.0)

  out_ref[0, 0] = _W_CE * loss_ce + _W_CEC * cec


def _combine(s0, labels_flat, gathered, cls_means):
  return pl.pallas_call(
      _combine_body,
      in_specs=[
          pl.BlockSpec(memory_space=pltpu.SMEM),
          pl.BlockSpec((1, _NP), lambda: (0, 0)),
          pl.BlockSpec((_NP, _D), lambda: (0, 0)),
          pl.BlockSpec((_NUM_KNOWN, _D), lambda: (0, 0)),
      ],
      out_specs=pl.BlockSpec(memory_space=pltpu.SMEM),
      out_shape=jax.ShapeDtypeStruct((1, 1), jnp.float32),
  )(s0, labels_flat, gathered, cls_means)


def kernel(pred_logits, object_embedding, cls_means, src_idx, labels):
  src_idx = src_idx.astype(jnp.int32)
  labels = labels.astype(jnp.int32)

  table = object_embedding.transpose(1, 0, 2).reshape(_Q * _B, _D)
  pred_cm = pred_logits.transpose(2, 0, 1)

  barange = jnp.arange(_B, dtype=jnp.int32)[:, None]
  eidx = jnp.zeros((_B, _TP), jnp.int32).at[:, :_T].set(
      src_idx * _B + barange).reshape(_NP)
  gathered = _get_sc_gather()(eidx, table)

  labels_flat = jnp.full((_B, _TP), -1, jnp.int32).at[:, :_T].set(labels)
  labels_flat = labels_flat.reshape(1, _NP)

  s0 = _dense_focal(pred_cm, src_idx, labels)
  total = _combine(s0, labels_flat, gathered, cls_means)
  return total[0, 0]


# speedup vs baseline: 1.0328x; 1.0328x over previous
import functools

import jax
import jax.numpy as jnp
from jax import lax
from jax.experimental import pallas as pl
from jax.experimental.pallas import tpu as pltpu
from jax.experimental.pallas import tpu_sc as plsc

_NUM_CLASSES = 81
_NUM_KNOWN = 80
_ALPHA_PROTO = 0.9
_TAU_CEC = 0.1
_FOCAL_ALPHA = 0.25
_W_CE = 2.0
_W_CEC = 0.5

_B, _Q, _D, _T = 32, 900, 256, 20
_N = _B * _T
_TP = 24
_NP = _B * _TP
_TX = 32
_NX = _B * _TX
_CCH = 9


def _make_sc_gather():
  mesh = plsc.VectorSubcoreMesh(core_axis_name="c", subcore_axis_name="s")
  info = plsc.get_sparse_core_info()
  num_cores = info.num_cores

  @functools.partial(
      pl.kernel,
      mesh=mesh,
      out_type=jax.ShapeDtypeStruct((_NP, _D), jnp.float32),
      scratch_types=[
          pltpu.VMEM((_TP,), jnp.int32),
          pltpu.VMEM((_TP, _D), jnp.float32),
          pltpu.SemaphoreType.DMA,
      ],
  )
  def sc_gather(eidx_hbm, table_hbm, emb_out, eidx_v, erows_v, sem_e):
    wid = lax.axis_index("s") * num_cores + lax.axis_index("c")
    ebase = pl.multiple_of(wid * _TP, 8)
    pltpu.sync_copy(eidx_hbm.at[pl.ds(ebase, _TP)], eidx_v)
    pltpu.async_copy(table_hbm.at[eidx_v], erows_v, sem_e).wait()
    pltpu.sync_copy(erows_v, emb_out.at[pl.ds(ebase, _TP)])

  return sc_gather


_sc_gather_cache = []


def _get_sc_gather():
  if not _sc_gather_cache:
    _sc_gather_cache.append(_make_sc_gather())
  return _sc_gather_cache[0]


_BCH = 8


def _dense_body(x_ref, si_ref, lb_ref, sv_ref, out_ref, acc_ref):
  i = pl.program_id(0)
  x = x_ref[...]

  a = jnp.exp(-jnp.abs(x))
  lg = jnp.log(1.0 + a)
  r = 1.0 / (1.0 + a)
  p = jnp.where(x >= 0.0, r, 1.0 - r)
  sp = jnp.maximum(x, 0.0) + lg
  s0 = jnp.sum(p * p * sp)

  q_iota = lax.broadcasted_iota(jnp.int32, (1, _Q), 1)
  t_iota = lax.broadcasted_iota(jnp.int32, (_BCH, 128), 1)
  b_iota = lax.broadcasted_iota(jnp.int32, (_BCH, 128), 0)
  xs = jnp.zeros((_BCH, 128), jnp.float32)
  for bb in range(_BCH):
    b = i * _BCH + bb
    sts = [si_ref[t, b] for t in range(_T)]
    lts = [lb_ref[t, b] for t in range(_T)]
    for t in range(_T):
      row = x_ref[pl.ds(lts[t], 1), pl.ds(bb, 1), :].reshape(1, _Q)
      xt = jnp.sum(jnp.where(q_iota == sts[t], row, 0.0))
      m2 = jnp.logical_and(t_iota == t, b_iota == bb)
      xs = jnp.where(m2, xt, xs)

  src = sv_ref[...]
  dup = jnp.zeros((_BCH, _T), jnp.float32)
  for d in range(1, _T):
    eq = (src[:, : _T - d] == src[:, d:]).astype(jnp.float32)
    eq = jnp.concatenate([eq, jnp.zeros((_BCH, d), jnp.float32)], axis=1)
    dup = jnp.maximum(dup, eq)
  fin = 1.0 - dup

  a2 = jnp.exp(-jnp.abs(xs))
  lg2 = jnp.log(1.0 + a2)
  r2 = 1.0 / (1.0 + a2)
  p2 = jnp.where(xs >= 0.0, r2, 1.0 - r2)
  sp2 = jnp.maximum(xs, 0.0) + lg2
  sn2 = sp2 - xs
  om = 1.0 - p2
  g = _FOCAL_ALPHA * om * om * sn2 - (1.0 - _FOCAL_ALPHA) * p2 * p2 * sp2
  part = (1.0 - _FOCAL_ALPHA) * s0 + jnp.sum(fin * g[:, :_T])

  @pl.when(i == 0)
  def _():
    acc_ref[0, 0] = 0.0

  acc = acc_ref[0, 0] + part
  acc_ref[0, 0] = acc

  @pl.when(i == _B // _BCH - 1)
  def _():
    out_ref[0, 0] = acc


def _dense_focal(pred_cm, src_t, labels_t, src_idx):
  return pl.pallas_call(
      _dense_body,
      grid=(_B // _BCH,),
      in_specs=[
          pl.BlockSpec((_NUM_CLASSES, _BCH, _Q), lambda i: (0, i, 0)),
          pl.BlockSpec(memory_space=pltpu.SMEM),
          pl.BlockSpec(memory_space=pltpu.SMEM),
          pl.BlockSpec((_BCH, _T), lambda i: (i, 0)),
      ],
      out_specs=pl.BlockSpec(memory_space=pltpu.SMEM),
      out_shape=jax.ShapeDtypeStruct((1, 1), jnp.float32),
      scratch_shapes=[pltpu.SMEM((1, 1), jnp.float32)],
      compiler_params=pltpu.CompilerParams(
          dimension_semantics=("arbitrary",)),
  )(pred_cm, src_t, labels_t, src_idx)


def _combine_body(s0_ref, lf_ref, g_ref, cm_ref, out_ref):
  num_boxes = jnp.maximum(jnp.float32(_N), 1.0)
  loss_ce = s0_ref[0, 0] / num_boxes

  embs = g_ref[...]
  nrm = jnp.sqrt(jnp.sum(embs * embs, axis=1, keepdims=True))
  embs_n = embs / jnp.maximum(nrm, 1e-6)

  lf = lf_ref[...]
  valid_col = lf >= 0
  k_iota = lax.broadcasted_iota(jnp.int32, (_NUM_KNOWN, _NP), 0)
  mask = (lf == k_iota).astype(jnp.float32)
  counts = jnp.sum(mask, axis=1, keepdims=True)

  sums = lax.dot_general(mask, embs, (((1,), (0,)), ((), ())),
                         preferred_element_type=jnp.float32)
  mean = sums / jnp.maximum(counts, 1.0)
  mnrm = jnp.sqrt(jnp.sum(mean * mean, axis=1, keepdims=True))
  mean_n = mean / jnp.maximum(mnrm, 1e-6)

  cm = cm_ref[...]
  upd = _ALPHA_PROTO * cm + (1.0 - _ALPHA_PROTO) * mean_n
  unrm = jnp.sqrt(jnp.sum(upd * upd, axis=1, keepdims=True))
  upd_n = upd / jnp.maximum(unrm, 1e-6)
  protos = jnp.where(counts > 0, upd_n, cm)
  pnrm = jnp.sqrt(jnp.sum(protos * protos, axis=1, keepdims=True))
  protos_n = protos / jnp.maximum(pnrm, 1e-6)

  sim = lax.dot_general(protos_n, embs_n, (((1,), (1,)), ((), ())),
                        preferred_element_type=jnp.float32)
  logits = jnp.where(valid_col, sim / _TAU_CEC, -1e30)
  m = jnp.max(logits, axis=1, keepdims=True)
  lse = jnp.log(jnp.sum(jnp.exp(logits - m), axis=1, keepdims=True)) + m
  logp = logits - lse
  per_class = -jnp.sum(logp * mask, axis=1, keepdims=True)
  per_class = per_class / jnp.maximum(counts, 1.0)
  validm = (counts > 0).astype(jnp.float32)
  cec = jnp.sum(per_class * validm) / jnp.maximum(jnp.sum(validm), 1.0)

  out_ref[0, 0] = _W_CE * loss_ce + _W_CEC * cec


def _combine(s0, labels_flat, gathered, cls_means):
  return pl.pallas_call(
      _combine_body,
      in_specs=[
          pl.BlockSpec(memory_space=pltpu.SMEM),
          pl.BlockSpec((1, _NP), lambda: (0, 0)),
          pl.BlockSpec((_NP, _D), lambda: (0, 0)),
          pl.BlockSpec((_NUM_KNOWN, _D), lambda: (0, 0)),
      ],
      out_specs=pl.BlockSpec(memory_space=pltpu.SMEM),
      out_shape=jax.ShapeDtypeStruct((1, 1), jnp.float32),
  )(s0, labels_flat, gathered, cls_means)


def kernel(pred_logits, object_embedding, cls_means, src_idx, labels):
  src_idx = src_idx.astype(jnp.int32)
  labels = labels.astype(jnp.int32)

  table = object_embedding.transpose(1, 0, 2).reshape(_Q * _B, _D)
  pred_cm = pred_logits.transpose(2, 0, 1)

  barange = jnp.arange(_B, dtype=jnp.int32)[:, None]
  eidx = jnp.zeros((_B, _TP), jnp.int32).at[:, :_T].set(
      src_idx * _B + barange).reshape(_NP)
  gathered = _get_sc_gather()(eidx, table)

  labels_flat = jnp.full((_B, _TP), -1, jnp.int32).at[:, :_T].set(labels)
  labels_flat = labels_flat.reshape(1, _NP)

  s0 = _dense_focal(pred_cm, src_idx.T, labels.T, src_idx)
  total = _combine(s0, labels_flat, gathered, cls_means)
  return total[0, 0]


# speedup vs baseline: 1.0835x; 1.0491x over previous
import functools

import jax
import jax.numpy as jnp
from jax import lax
from jax.experimental import pallas as pl
from jax.experimental.pallas import tpu as pltpu
from jax.experimental.pallas import tpu_sc as plsc

_NUM_CLASSES = 81
_NUM_KNOWN = 80
_ALPHA_PROTO = 0.9
_TAU_CEC = 0.1
_FOCAL_ALPHA = 0.25
_W_CE = 2.0
_W_CEC = 0.5

_B, _Q, _D, _T = 32, 900, 256, 20
_N = _B * _T
_TP = 24
_NP = _B * _TP
_TX = 32
_NX = _B * _TX
_CCH = 9


def _make_sc_gather():
  mesh = plsc.VectorSubcoreMesh(core_axis_name="c", subcore_axis_name="s")
  info = plsc.get_sparse_core_info()
  num_cores = info.num_cores

  @functools.partial(
      pl.kernel,
      mesh=mesh,
      out_type=jax.ShapeDtypeStruct((_NP, _D), jnp.float32),
      scratch_types=[
          pltpu.VMEM((_TP,), jnp.int32),
          pltpu.VMEM((_TP, _D), jnp.float32),
          pltpu.SemaphoreType.DMA,
      ],
  )
  def sc_gather(eidx_hbm, table_hbm, emb_out, eidx_v, erows_v, sem_e):
    wid = lax.axis_index("s") * num_cores + lax.axis_index("c")
    ebase = pl.multiple_of(wid * _TP, 8)
    pltpu.sync_copy(eidx_hbm.at[pl.ds(ebase, _TP)], eidx_v)
    pltpu.async_copy(table_hbm.at[eidx_v], erows_v, sem_e).wait()
    pltpu.sync_copy(erows_v, emb_out.at[pl.ds(ebase, _TP)])

  return sc_gather


_sc_gather_cache = []


def _get_sc_gather():
  if not _sc_gather_cache:
    _sc_gather_cache.append(_make_sc_gather())
  return _sc_gather_cache[0]


_BCH = 8


def _dense_body(x_ref, si_ref, lb_ref, sv_ref, out_ref, acc_ref):
  i = pl.program_id(0)
  x = x_ref[...]

  a = jnp.exp(-jnp.abs(x))
  lg = jnp.log(1.0 + a)
  r = 1.0 / (1.0 + a)
  p = jnp.where(x >= 0.0, r, 1.0 - r)
  sp = jnp.maximum(x, 0.0) + lg
  s0 = jnp.sum(p * p * sp)

  q_iota = lax.broadcasted_iota(jnp.int32, (1, _Q), 1)
  t_iota = lax.broadcasted_iota(jnp.int32, (_BCH, 128), 1)
  b_iota = lax.broadcasted_iota(jnp.int32, (_BCH, 128), 0)
  xs = jnp.zeros((_BCH, 128), jnp.float32)
  for bb in range(_BCH):
    b = i * _BCH + bb
    sts = [si_ref[t, b] for t in range(_T)]
    lts = [lb_ref[t, b] for t in range(_T)]
    for t in range(_T):
      row = x_ref[pl.ds(lts[t], 1), pl.ds(bb, 1), :].reshape(1, _Q)
      xt = jnp.sum(jnp.where(q_iota == sts[t], row, 0.0))
      m2 = jnp.logical_and(t_iota == t, b_iota == bb)
      xs = jnp.where(m2, xt, xs)

  src = sv_ref[...]
  dup = jnp.zeros((_BCH, _T), jnp.float32)
  for d in range(1, _T):
    eq = (src[:, : _T - d] == src[:, d:]).astype(jnp.float32)
    eq = jnp.concatenate([eq, jnp.zeros((_BCH, d), jnp.float32)], axis=1)
    dup = jnp.maximum(dup, eq)
  fin = 1.0 - dup

  a2 = jnp.exp(-jnp.abs(xs))
  lg2 = jnp.log(1.0 + a2)
  r2 = 1.0 / (1.0 + a2)
  p2 = jnp.where(xs >= 0.0, r2, 1.0 - r2)
  sp2 = jnp.maximum(xs, 0.0) + lg2
  sn2 = sp2 - xs
  om = 1.0 - p2
  g = _FOCAL_ALPHA * om * om * sn2 - (1.0 - _FOCAL_ALPHA) * p2 * p2 * sp2
  part = (1.0 - _FOCAL_ALPHA) * s0 + jnp.sum(fin * g[:, :_T])

  @pl.when(i == 0)
  def _():
    acc_ref[0, 0] = 0.0

  acc = acc_ref[0, 0] + part
  acc_ref[0, 0] = acc

  @pl.when(i == _B // _BCH - 1)
  def _():
    out_ref[0, 0] = acc


def _dense_focal(pred_cm, src_t, labels_t, src_idx):
  return pl.pallas_call(
      _dense_body,
      grid=(_B // _BCH,),
      in_specs=[
          pl.BlockSpec((_NUM_CLASSES, _BCH, _Q), lambda i: (0, i, 0)),
          pl.BlockSpec(memory_space=pltpu.SMEM),
          pl.BlockSpec(memory_space=pltpu.SMEM),
          pl.BlockSpec((_BCH, _T), lambda i: (i, 0)),
      ],
      out_specs=pl.BlockSpec(memory_space=pltpu.SMEM),
      out_shape=jax.ShapeDtypeStruct((1, 1), jnp.float32),
      scratch_shapes=[pltpu.SMEM((1, 1), jnp.float32)],
      compiler_params=pltpu.CompilerParams(
          dimension_semantics=("arbitrary",)),
  )(pred_cm, src_t, labels_t, src_idx)


def _combine_body(s0_ref, lf_ref, g_ref, cm_ref, out_ref):
  num_boxes = jnp.maximum(jnp.float32(_N), 1.0)
  loss_ce = s0_ref[0, 0] / num_boxes

  embs = g_ref[...]
  nrm = jnp.sqrt(jnp.sum(embs * embs, axis=1, keepdims=True))
  embs_n = embs / jnp.maximum(nrm, 1e-6)

  lf = lf_ref[...]
  valid_col = lf >= 0
  k_iota = lax.broadcasted_iota(jnp.int32, (_NUM_KNOWN, _NP), 0)
  mask = (lf == k_iota).astype(jnp.float32)
  counts = jnp.sum(mask, axis=1, keepdims=True)

  sums = lax.dot_general(mask, embs, (((1,), (0,)), ((), ())),
                         preferred_element_type=jnp.float32)
  mean = sums / jnp.maximum(counts, 1.0)
  mnrm = jnp.sqrt(jnp.sum(mean * mean, axis=1, keepdims=True))
  mean_n = mean / jnp.maximum(mnrm, 1e-6)

  cm = cm_ref[...]
  upd = _ALPHA_PROTO * cm + (1.0 - _ALPHA_PROTO) * mean_n
  unrm = jnp.sqrt(jnp.sum(upd * upd, axis=1, keepdims=True))
  upd_n = upd / jnp.maximum(unrm, 1e-6)
  protos = jnp.where(counts > 0, upd_n, cm)
  pnrm = jnp.sqrt(jnp.sum(protos * protos, axis=1, keepdims=True))
  protos_n = protos / jnp.maximum(pnrm, 1e-6)

  sim = lax.dot_general(protos_n, embs_n, (((1,), (1,)), ((), ())),
                        preferred_element_type=jnp.float32)
  logits = jnp.where(valid_col, sim / _TAU_CEC, -1e30)
  m = jnp.max(logits, axis=1, keepdims=True)
  lse = jnp.log(jnp.sum(jnp.exp(logits - m), axis=1, keepdims=True)) + m
  logp = logits - lse
  per_class = -jnp.sum(logp * mask, axis=1, keepdims=True)
  per_class = per_class / jnp.maximum(counts, 1.0)
  validm = (counts > 0).astype(jnp.float32)
  cec = jnp.sum(per_class * validm) / jnp.maximum(jnp.sum(validm), 1.0)

  out_ref[0, 0] = _W_CE * loss_ce + _W_CEC * cec


def _combine(s0, labels_flat, gathered, cls_means):
  return pl.pallas_call(
      _combine_body,
      in_specs=[
          pl.BlockSpec(memory_space=pltpu.SMEM),
          pl.BlockSpec((1, _NP), lambda: (0, 0)),
          pl.BlockSpec((_NP, _D), lambda: (0, 0)),
          pl.BlockSpec((_NUM_KNOWN, _D), lambda: (0, 0)),
      ],
      out_specs=pl.BlockSpec(memory_space=pltpu.SMEM),
      out_shape=jax.ShapeDtypeStruct((1, 1), jnp.float32),
  )(s0, labels_flat, gathered, cls_means)


def kernel(pred_logits, object_embedding, cls_means, src_idx, labels):
  src_idx = src_idx.astype(jnp.int32)
  labels = labels.astype(jnp.int32)

  table = object_embedding.transpose(1, 0, 2).reshape(_Q * _B, _D)
  pred_cm = pred_logits.transpose(2, 0, 1)

  barange = jnp.arange(_B, dtype=jnp.int32)[:, None]
  eidx = jnp.pad(src_idx * _B + barange, ((0, 0), (0, _TP - _T))).reshape(_NP)
  gathered = _get_sc_gather()(eidx, table)

  labels_flat = jnp.pad(labels, ((0, 0), (0, _TP - _T)),
                        constant_values=-1).reshape(1, _NP)

  s0 = _dense_focal(pred_cm, src_idx.T, labels.T, src_idx)
  total = _combine(s0, labels_flat, gathered, cls_means)
  return total[0, 0]
